# baseline (speedup 1.0000x reference)
import functools

import jax
import jax.numpy as jnp
from jax import lax
from jax.experimental import pallas as pl
from jax.experimental.pallas import tpu as pltpu
from jax.experimental.pallas import tpu_sc as plsc

_B = 16384
_D = 64
_NC = 2
_NS = 16
_NW = _NC * _NS
_RPT = _B // _NW
_CH = 128
_NCHUNK = _RPT // _CH

_mesh = plsc.VectorSubcoreMesh(core_axis_name="c", subcore_axis_name="s")


@functools.partial(
    pl.kernel,
    mesh=_mesh,
    compiler_params=pltpu.CompilerParams(
        needs_layout_passes=False, use_tc_tiling_on_sc=False
    ),
    out_type=jax.ShapeDtypeStruct((_B,), jnp.float32),
    scratch_types=[
        pltpu.VMEM((_NCHUNK, _CH), jnp.int32),
        pltpu.VMEM((_NCHUNK, _CH), jnp.int32),
        pltpu.VMEM((_NCHUNK, _CH), jnp.int32),
        pltpu.VMEM((_RPT, _D), jnp.float32),
        pltpu.VMEM((_RPT, _D), jnp.float32),
        pltpu.VMEM((_RPT, _D), jnp.float32),
        pltpu.VMEM((_RPT,), jnp.float32),
        pltpu.SemaphoreType.DMA,
    ],
)
def _sc_scores(uid_hbm, pid_hbm, nid_hbm, utab_hbm, itab_hbm, out_hbm,
               idx_u, idx_p, idx_n, rows_u, rows_p, rows_n, tmp_v, sem):
    wid = lax.axis_index("s") * _NC + lax.axis_index("c")
    base = wid * _RPT

    for j in range(_NCHUNK):
        off = base + j * _CH
        pltpu.sync_copy(uid_hbm.at[pl.ds(off, _CH)], idx_u.at[j])
        pltpu.sync_copy(pid_hbm.at[pl.ds(off, _CH)], idx_p.at[j])
        pltpu.sync_copy(nid_hbm.at[pl.ds(off, _CH)], idx_n.at[j])

    copies = []
    for j in range(_NCHUNK):
        r = pl.ds(j * _CH, _CH)
        copies.append(pltpu.async_copy(utab_hbm.at[idx_u.at[j]], rows_u.at[r], sem))
        copies.append(pltpu.async_copy(itab_hbm.at[idx_p.at[j]], rows_p.at[r], sem))
        copies.append(pltpu.async_copy(itab_hbm.at[idx_n.at[j]], rows_n.at[r], sem))
    for c in copies:
        c.wait()

    lane = lax.iota(jnp.int32, 16)

    def body(g, carry):
        tvec = jnp.zeros((16,), jnp.float32)
        for l in range(16):
            r = g * 16 + l
            acc = jnp.zeros((16,), jnp.float32)
            for k in range(_D // 16):
                sl = pl.ds(k * 16, 16)
                u = rows_u[r, sl]
                p = rows_p[r, sl]
                n = rows_n[r, sl]
                acc = acc + u * (p - n)
            tvec = jnp.where(lane == l, jnp.sum(acc), tvec)
        tmp_v[pl.ds(g * 16, 16)] = tvec
        return carry

    lax.fori_loop(0, _RPT // 16, body, 0)
    pltpu.sync_copy(tmp_v, out_hbm.at[pl.ds(base, _RPT)])


def _loss_body(x_ref, o_ref):
    x = x_ref[...]
    z = jnp.exp(-jnp.abs(x))
    ls = jnp.minimum(x, 0.0) - jnp.log(1.0 + z)
    o_ref[0, 0] = -jnp.sum(ls)


def kernel(user_id, pos_id, neg_id, user_table, item_table):
    tmp = _sc_scores(user_id, pos_id, neg_id, user_table, item_table)
    loss = pl.pallas_call(
        _loss_body,
        out_shape=jax.ShapeDtypeStruct((1, 1), jnp.float32),
        out_specs=pl.BlockSpec(memory_space=pltpu.SMEM),
    )(tmp.reshape(128, 128))
    return loss[0, 0]
